# Initial kernel scaffold; baseline (speedup 1.0000x reference)
import jax
import jax.numpy as jnp
from jax.experimental import pallas as pl


def kernel(x, W1, b1, W2, b2, Wg1, bg1, Wg2, bg2, Wc, bc):
    raise NotImplementedError("write your pallas kernel here")


# speedup vs baseline: 1.1720x; 1.1720x over previous
import functools

import jax
import jax.numpy as jnp
from jax.experimental import pallas as pl


def _moe_body(x_ref, W1_ref, b1_ref, W2_ref, b2_ref,
              Wg1_ref, bg1_ref, Wg2_ref, bg2_ref, Wc_ref, bc_ref,
              out_ref, scores_ref, *, n_experts):
    xblk = x_ref[...]

    g = jnp.dot(xblk, Wg1_ref[...], preferred_element_type=jnp.float32)
    g = jnp.maximum(g + bg1_ref[...], 0.0)
    logits = jnp.dot(g.astype(jnp.bfloat16), Wg2_ref[...],
                     preferred_element_type=jnp.float32) + bg2_ref[...]
    m = jnp.max(logits, axis=1, keepdims=True)
    ex = jnp.exp(logits - m)
    scores_ref[...] = ex / jnp.sum(ex, axis=1, keepdims=True)

    Wc = Wc_ref[...]
    bc = bc_ref[...]
    for e in range(n_experts):
        h = jnp.dot(xblk, W1_ref[e], preferred_element_type=jnp.float32)
        h = jnp.maximum(h + b1_ref[e:e + 1, :], 0.0)
        eo = jnp.dot(h.astype(jnp.bfloat16), W2_ref[e],
                     preferred_element_type=jnp.float32) + b2_ref[e:e + 1, :]
        o = jnp.dot(eo.astype(jnp.bfloat16), Wc,
                    preferred_element_type=jnp.float32) + bc
        out_ref[:, e, :] = o


def kernel(x, W1, b1, W2, b2, Wg1, bg1, Wg2, bg2, Wc, bc):
    B, D = x.shape
    E, _, H = W1.shape
    C = Wc.shape[1]
    BT = 256 if B % 256 == 0 else B

    bf = jnp.bfloat16
    xb = x.astype(bf)
    W1b, W2b = W1.astype(bf), W2.astype(bf)
    Wg1b, Wg2b, Wcb = Wg1.astype(bf), Wg2.astype(bf), Wc.astype(bf)
    bg1_2 = bg1.reshape(1, D)
    bg2_2 = bg2.reshape(1, E)
    bc_2 = bc.reshape(1, C)

    whole = lambda *dims: pl.BlockSpec(dims, lambda t: (0,) * len(dims))
    grid_spec = pl.GridSpec(
        grid=(B // BT,),
        in_specs=[
            pl.BlockSpec((BT, D), lambda t: (t, 0)),
            whole(E, D, H),
            whole(E, H),
            whole(E, H, H),
            whole(E, H),
            whole(D, D),
            whole(1, D),
            whole(D, E),
            whole(1, E),
            whole(H, C),
            whole(1, C),
        ],
        out_specs=[
            pl.BlockSpec((BT, E, C), lambda t: (t, 0, 0)),
            pl.BlockSpec((BT, E), lambda t: (t, 0)),
        ],
    )
    out, scores = pl.pallas_call(
        functools.partial(_moe_body, n_experts=E),
        grid_spec=grid_spec,
        out_shape=[
            jax.ShapeDtypeStruct((B, E, C), jnp.float32),
            jax.ShapeDtypeStruct((B, E), jnp.float32),
        ],
    )(xb, W1b, b1, W2b, b2, Wg1b, bg1_2, Wg2b, bg2_2, Wcb, bc_2)
    return (out, scores)


# speedup vs baseline: 1.2234x; 1.0439x over previous
import functools

import jax
import jax.numpy as jnp
from jax.experimental import pallas as pl


def _fold_body(W2_ref, Wc_ref, b2_ref, bc_ref, W2c_ref, bc2_ref):
    e = pl.program_id(0)
    prod = jnp.dot(W2_ref[0], Wc_ref[...], preferred_element_type=jnp.float32)
    W2c_ref[0] = prod.astype(jnp.bfloat16)
    b2row = b2_ref[pl.ds(e, 1), :].astype(jnp.bfloat16)
    bc2_ref[0] = jnp.dot(b2row, Wc_ref[...],
                         preferred_element_type=jnp.float32) + bc_ref[...]


def _moe_body(x_ref, W1_ref, b1_ref, W2c_ref, bc2_ref,
              Wg1_ref, bg1_ref, Wg2_ref, bg2_ref,
              out_ref, scores_ref, *, n_experts):
    xblk = x_ref[...]

    g = jnp.dot(xblk, Wg1_ref[...], preferred_element_type=jnp.float32)
    g = jnp.maximum(g + bg1_ref[...], 0.0)
    logits = jnp.dot(g.astype(jnp.bfloat16), Wg2_ref[...],
                     preferred_element_type=jnp.float32) + bg2_ref[...]
    m = jnp.max(logits, axis=1, keepdims=True)
    ex = jnp.exp(logits - m)
    scores_ref[...] = ex / jnp.sum(ex, axis=1, keepdims=True)

    for e in range(n_experts):
        h = jnp.dot(xblk, W1_ref[e], preferred_element_type=jnp.float32)
        h = jnp.maximum(h + b1_ref[e:e + 1, :], 0.0)
        o = jnp.dot(h.astype(jnp.bfloat16), W2c_ref[e],
                    preferred_element_type=jnp.float32) + bc2_ref[e]
        out_ref[:, e, :] = o


def kernel(x, W1, b1, W2, b2, Wg1, bg1, Wg2, bg2, Wc, bc):
    B, D = x.shape
    E, _, H = W1.shape
    C = Wc.shape[1]
    BT = 256 if B % 256 == 0 else B

    bf = jnp.bfloat16
    xb = x.astype(bf)
    W1b, W2b = W1.astype(bf), W2.astype(bf)
    Wg1b, Wg2b, Wcb = Wg1.astype(bf), Wg2.astype(bf), Wc.astype(bf)
    bg1_2 = bg1.reshape(1, D)
    bg2_2 = bg2.reshape(1, E)
    bc_2 = bc.reshape(1, C)

    W2c, bc2 = pl.pallas_call(
        _fold_body,
        grid=(E,),
        in_specs=[
            pl.BlockSpec((1, H, H), lambda e: (e, 0, 0)),
            pl.BlockSpec((H, C), lambda e: (0, 0)),
            pl.BlockSpec((E, H), lambda e: (0, 0)),
            pl.BlockSpec((1, C), lambda e: (0, 0)),
        ],
        out_specs=[
            pl.BlockSpec((1, H, C), lambda e: (e, 0, 0)),
            pl.BlockSpec((1, 1, C), lambda e: (e, 0, 0)),
        ],
        out_shape=[
            jax.ShapeDtypeStruct((E, H, C), bf),
            jax.ShapeDtypeStruct((E, 1, C), jnp.float32),
        ],
    )(W2b, Wcb, b2, bc_2)

    whole = lambda *dims: pl.BlockSpec(dims, lambda t: (0,) * len(dims))
    out, scores = pl.pallas_call(
        functools.partial(_moe_body, n_experts=E),
        grid=(B // BT,),
        in_specs=[
            pl.BlockSpec((BT, D), lambda t: (t, 0)),
            whole(E, D, H),
            whole(E, H),
            whole(E, H, C),
            whole(E, 1, C),
            whole(D, D),
            whole(1, D),
            whole(D, E),
            whole(1, E),
        ],
        out_specs=[
            pl.BlockSpec((BT, E, C), lambda t: (t, 0, 0)),
            pl.BlockSpec((BT, E), lambda t: (t, 0)),
        ],
        out_shape=[
            jax.ShapeDtypeStruct((B, E, C), jnp.float32),
            jax.ShapeDtypeStruct((B, E), jnp.float32),
        ],
    )(xb, W1b, b1, W2c, bc2, Wg1b, bg1_2, Wg2b, bg2_2)
    return (out, scores)
